# baseline (speedup 1.0000x reference)
import functools

import jax
import jax.numpy as jnp
from jax import lax
from jax.experimental import pallas as pl
from jax.experimental.pallas import tpu as pltpu
from jax.experimental.pallas import tpu_sc as plsc

N = 10000
F = 128
NCLS = 40
NPAD = 10240
E = 320000
NCORES = 2
NSUB = 16
NTILES = NCORES * NSUB
EPT = 10240
EPAD = EPT * NTILES
CHUNK = 128
NCHUNKS = EPT // CHUNK
SCHUNK = 32
SNCHUNKS = EPT // SCHUNK
SNB = 8
RPT = NPAD // NSUB
DUMP = NPAD - 1
BLK = 1024
GRID = NPAD // BLK

NB = 4

_mesh = plsc.VectorSubcoreMesh(
    core_axis_name="c", subcore_axis_name="s", num_cores=NCORES,
    num_subcores=NSUB)


LAST_REAL = E - (NTILES - 1) * EPT


@functools.partial(
    pl.kernel,
    out_type=jax.ShapeDtypeStruct((NCORES, NPAD), jnp.float32),
    mesh=_mesh,
    scratch_types=[
        pltpu.VMEM((CHUNK,), jnp.int32),
        pltpu.VMEM((CHUNK,), jnp.int32),
        pltpu.VMEM((CHUNK,), jnp.int32),
        pltpu.VMEM((CHUNK,), jnp.int32),
        pltpu.VMEM((CHUNK,), jnp.float32),
        pltpu.VMEM((RPT,), jnp.float32),
        pltpu.VMEM_SHARED((NPAD,), jnp.float32),
        pltpu.SemaphoreType.DMA,
        pltpu.SemaphoreType.DMA,
        pltpu.SemaphoreType.DMA,
        pltpu.SemaphoreType.DMA,
        pltpu.SemaphoreType.DMA,
        pltpu.SemaphoreType.DMA,
        pltpu.SemaphoreType.DMA,
        pltpu.SemaphoreType.DMA,
    ],
)
def _hist(ei_hbm, degp_hbm, d0, d1, d2, d3, ones, zbuf, acc,
          i0, i1, i2, i3, s0, s1, s2, s3):
    didx = (d0, d1, d2, d3)
    isem = (i0, i1, i2, i3)
    ssem = (s0, s1, s2, s3)
    c = lax.axis_index("c")
    s = lax.axis_index("s")
    wid = c * NSUB + s

    def _fill_z(i, _):
        zbuf[pl.ds(i * 16, 16)] = jnp.zeros((16,), jnp.float32)
        return 0
    lax.fori_loop(0, RPT // 16, _fill_z, 0)
    for j in range(CHUNK // 16):
        ones[pl.ds(j * 16, 16)] = jnp.ones((16,), jnp.float32)
    pltpu.sync_copy(zbuf, acc.at[pl.ds(s * RPT, RPT)])
    plsc.subcore_barrier()

    ebase = wid * EPT
    ngrp = jnp.where(wid == NTILES - 1, LAST_REAL // CHUNK // NB,
                     NCHUNKS // NB)

    def _grp(t, _):
        for b in range(NB):
            j = t * NB + b

            @pl.when(t > 0)
            def _():
                pltpu.make_async_copy(
                    ei_hbm.at[1, pl.ds(0, CHUNK)], didx[b], ssem[b]).wait()
            pltpu.async_copy(
                ei_hbm.at[1, pl.ds(ebase + j * CHUNK, CHUNK)], didx[b], isem[b])
        for b in range(NB):
            pltpu.make_async_copy(
                ei_hbm.at[1, pl.ds(0, CHUNK)], didx[b], isem[b]).wait()
            pltpu.async_copy(ones, acc.at[didx[b]], ssem[b], add=True)
        return 0
    lax.fori_loop(0, ngrp, _grp, 0)
    for b in range(NB):
        pltpu.make_async_copy(
            ei_hbm.at[1, pl.ds(0, CHUNK)], didx[b], ssem[b]).wait()
    plsc.subcore_barrier()
    pltpu.sync_copy(acc.at[pl.ds(s * RPT, RPT)],
                    degp_hbm.at[c, pl.ds(s * RPT, RPT)])


@functools.partial(
    pl.kernel,
    out_type=jax.ShapeDtypeStruct((NCORES, NPAD, F), jnp.float32),
    mesh=_mesh,
    scratch_types=[
        pltpu.VMEM((EPT,), jnp.int32),
        pltpu.VMEM((SCHUNK,), jnp.int32),
        pltpu.VMEM((SCHUNK,), jnp.int32),
        pltpu.VMEM((SCHUNK,), jnp.int32),
        pltpu.VMEM((SCHUNK,), jnp.int32),
        pltpu.VMEM((SCHUNK,), jnp.int32),
        pltpu.VMEM((SCHUNK,), jnp.int32),
        pltpu.VMEM((SCHUNK,), jnp.int32),
        pltpu.VMEM((SCHUNK,), jnp.int32),
        pltpu.VMEM((SCHUNK, F), jnp.float32),
        pltpu.VMEM((SCHUNK, F), jnp.float32),
        pltpu.VMEM((SCHUNK, F), jnp.float32),
        pltpu.VMEM((SCHUNK, F), jnp.float32),
        pltpu.VMEM((SCHUNK, F), jnp.float32),
        pltpu.VMEM((SCHUNK, F), jnp.float32),
        pltpu.VMEM((SCHUNK, F), jnp.float32),
        pltpu.VMEM((SCHUNK, F), jnp.float32),
        pltpu.VMEM_SHARED((NPAD, F), jnp.float32),
    ] + [pltpu.SemaphoreType.DMA] * 24,
)
def _scatter(g_hbm, ei_hbm, sout_hbm, sidx_all,
             d0, d1, d2, d3, d4, d5, d6, d7,
             r0, r1, r2, r3, r4, r5, r6, r7, acc,
             i0, i1, i2, i3, i4, i5, i6, i7,
             g0, g1, g2, g3, g4, g5, g6, g7,
             s0, s1, s2, s3, s4, s5, s6, s7):
    didx = (d0, d1, d2, d3, d4, d5, d6, d7)
    rows = (r0, r1, r2, r3, r4, r5, r6, r7)
    isem = (i0, i1, i2, i3, i4, i5, i6, i7)
    gsem = (g0, g1, g2, g3, g4, g5, g6, g7)
    ssem = (s0, s1, s2, s3, s4, s5, s6, s7)
    c = lax.axis_index("c")
    s = lax.axis_index("s")
    wid = c * NSUB + s

    def _zrow(i, _):
        for jj in range(F // 16):
            r0[i, pl.ds(jj * 16, 16)] = jnp.zeros((16,), jnp.float32)
        return 0
    lax.fori_loop(0, SCHUNK, _zrow, 0)
    for k in range(RPT // SCHUNK):
        pltpu.sync_copy(r0, acc.at[pl.ds(s * RPT + k * SCHUNK, SCHUNK)])
    plsc.subcore_barrier()

    ebase = wid * EPT
    last = wid == NTILES - 1
    pltpu.sync_copy(
        ei_hbm.at[0, pl.ds(jnp.where(last, E - EPT, ebase), EPT)], sidx_all)
    soff = jnp.where(last, EPT - LAST_REAL, 0)
    ngrp = jnp.where(last, LAST_REAL // SCHUNK // SNB, SNCHUNKS // SNB)

    def _grp(t, _):
        for b in range(SNB):
            j = t * SNB + b

            @pl.when(t > 0)
            def _():
                pltpu.make_async_copy(
                    g_hbm.at[pl.ds(0, SCHUNK)], rows[b], ssem[b]).wait()
            pltpu.async_copy(
                ei_hbm.at[1, pl.ds(ebase + j * SCHUNK, SCHUNK)], didx[b], isem[b])
            pltpu.async_copy(
                g_hbm.at[sidx_all.at[pl.ds(soff + j * SCHUNK, SCHUNK)]],
                rows[b], gsem[b])
        for b in range(SNB):
            j = t * SNB + b
            pltpu.make_async_copy(
                ei_hbm.at[1, pl.ds(0, SCHUNK)], didx[b], isem[b]).wait()
            pltpu.make_async_copy(
                g_hbm.at[pl.ds(0, SCHUNK)], rows[b], gsem[b]).wait()
            pltpu.async_copy(rows[b], acc.at[didx[b]], ssem[b], add=True)
        return 0
    lax.fori_loop(0, ngrp, _grp, 0)
    for b in range(SNB):
        pltpu.make_async_copy(
            g_hbm.at[pl.ds(0, SCHUNK)], rows[b], ssem[b]).wait()
    plsc.subcore_barrier()
    pltpu.sync_copy(acc.at[pl.ds(s * RPT, RPT)],
                    sout_hbm.at[c, pl.ds(s * RPT, RPT)])


def _dinv_block(degp_ref, i):
    d = degp_ref[0, pl.ds(i * BLK, BLK)] + degp_ref[1, pl.ds(i * BLK, BLK)]
    return lax.rsqrt(d + 1.0)


def _mm_raw_body(x_ref, w_ref, h_ref):
    h_ref[...] = jnp.dot(x_ref[...], w_ref[...],
                         preferred_element_type=jnp.float32)


def _scale_body(degp_ref, h_ref, g_ref):
    i = pl.program_id(0)
    dinv = _dinv_block(degp_ref, i)
    g_ref[...] = h_ref[...] * dinv[:, None]


def _combine_mm_body(degp_ref, s_ref, g_ref, b_ref, w_ref, g2_ref):
    i = pl.program_id(0)
    dinv = _dinv_block(degp_ref, i)
    t = s_ref[0] + s_ref[1] + g_ref[...]
    z = jnp.maximum(t * dinv[:, None] + b_ref[...], 0.0)
    h2 = jnp.dot(z, w_ref[...], preferred_element_type=jnp.float32)
    g2_ref[...] = h2 * dinv[:, None]


def _final_body(degp_ref, s_ref, g_ref, b_ref, w_ref, fcb_ref, o_ref):
    i = pl.program_id(0)
    dinv = _dinv_block(degp_ref, i)
    t = s_ref[0] + s_ref[1] + g_ref[...]
    z = jnp.maximum(t * dinv[:, None] + b_ref[...], 0.0)
    o_ref[...] = (jnp.dot(z, w_ref[...], preferred_element_type=jnp.float32)
                  + fcb_ref[...])


_degp_spec = pl.BlockSpec((NCORES, NPAD), lambda i: (0, 0))
_row_spec = pl.BlockSpec((BLK, F), lambda i: (i, 0))
_s_spec = pl.BlockSpec((NCORES, BLK, F), lambda i: (0, i, 0))
_w_spec = pl.BlockSpec((F, F), lambda i: (0, 0))
_b_spec = pl.BlockSpec((1, F), lambda i: (0, 0))

_mm_raw = pl.pallas_call(
    _mm_raw_body,
    grid=(GRID,),
    in_specs=[_row_spec, _w_spec],
    out_specs=_row_spec,
    out_shape=jax.ShapeDtypeStruct((NPAD, F), jnp.float32),
)

_scale = pl.pallas_call(
    _scale_body,
    grid=(GRID,),
    in_specs=[_degp_spec, _row_spec],
    out_specs=_row_spec,
    out_shape=jax.ShapeDtypeStruct((NPAD, F), jnp.float32),
)

_combine_mm = pl.pallas_call(
    _combine_mm_body,
    grid=(GRID,),
    in_specs=[_degp_spec, _s_spec, _row_spec, _b_spec, _w_spec],
    out_specs=_row_spec,
    out_shape=jax.ShapeDtypeStruct((NPAD, F), jnp.float32),
)

_final = pl.pallas_call(
    _final_body,
    grid=(GRID,),
    in_specs=[_degp_spec, _s_spec, _row_spec, _b_spec,
              pl.BlockSpec((F, NCLS), lambda i: (0, 0)),
              pl.BlockSpec((1, NCLS), lambda i: (0, 0))],
    out_specs=pl.BlockSpec((BLK, NCLS), lambda i: (i, 0)),
    out_shape=jax.ShapeDtypeStruct((N, NCLS), jnp.float32),
)


def kernel(x, edge_index, W1, b1, W2, b2, fcW, fcb):
    degp = _hist(edge_index)
    h1 = _mm_raw(x, W1)
    g1 = _scale(degp, h1)
    s1 = _scatter(g1, edge_index)
    g2 = _combine_mm(degp, s1, g1, b1.reshape(1, F), W2)
    s2 = _scatter(g2, edge_index)
    return _final(degp, s2, g2, b2.reshape(1, F), fcW, fcb.reshape(1, NCLS))


# speedup vs baseline: 1.0030x; 1.0030x over previous
import functools

import jax
import jax.numpy as jnp
from jax import lax
from jax.experimental import pallas as pl
from jax.experimental.pallas import tpu as pltpu
from jax.experimental.pallas import tpu_sc as plsc

N = 10000
F = 128
NCLS = 40
NPAD = 10240
E = 320000
NCORES = 2
NSUB = 16
NTILES = NCORES * NSUB
EPT = 10240
EPAD = EPT * NTILES
CHUNK = 128
NCHUNKS = EPT // CHUNK
SCHUNK = 32
SNCHUNKS = EPT // SCHUNK
SNB = 8
RPT = NPAD // NSUB
DUMP = NPAD - 1
BLK = 1024
GRID = NPAD // BLK

NB = 4

_mesh = plsc.VectorSubcoreMesh(
    core_axis_name="c", subcore_axis_name="s", num_cores=NCORES,
    num_subcores=NSUB)


LAST_REAL = E - (NTILES - 1) * EPT


@functools.partial(
    pl.kernel,
    out_type=jax.ShapeDtypeStruct((NCORES, NPAD), jnp.float32),
    mesh=_mesh,
    scratch_types=[
        pltpu.VMEM((CHUNK,), jnp.int32),
        pltpu.VMEM((CHUNK,), jnp.int32),
        pltpu.VMEM((CHUNK,), jnp.int32),
        pltpu.VMEM((CHUNK,), jnp.int32),
        pltpu.VMEM((CHUNK,), jnp.float32),
        pltpu.VMEM((RPT,), jnp.float32),
        pltpu.VMEM_SHARED((NPAD,), jnp.float32),
        pltpu.SemaphoreType.DMA,
        pltpu.SemaphoreType.DMA,
        pltpu.SemaphoreType.DMA,
        pltpu.SemaphoreType.DMA,
        pltpu.SemaphoreType.DMA,
        pltpu.SemaphoreType.DMA,
        pltpu.SemaphoreType.DMA,
        pltpu.SemaphoreType.DMA,
    ],
)
def _hist(ei_hbm, degp_hbm, d0, d1, d2, d3, ones, zbuf, acc,
          i0, i1, i2, i3, s0, s1, s2, s3):
    didx = (d0, d1, d2, d3)
    isem = (i0, i1, i2, i3)
    ssem = (s0, s1, s2, s3)
    c = lax.axis_index("c")
    s = lax.axis_index("s")
    wid = c * NSUB + s

    def _fill_z(i, _):
        zbuf[pl.ds(i * 16, 16)] = jnp.zeros((16,), jnp.float32)
        return 0
    lax.fori_loop(0, RPT // 16, _fill_z, 0)
    for j in range(CHUNK // 16):
        ones[pl.ds(j * 16, 16)] = jnp.ones((16,), jnp.float32)
    pltpu.sync_copy(zbuf, acc.at[pl.ds(s * RPT, RPT)])
    plsc.subcore_barrier()

    ebase = wid * EPT
    ngrp = jnp.where(wid == NTILES - 1, LAST_REAL // CHUNK // NB,
                     NCHUNKS // NB)

    def _grp(t, _):
        for b in range(NB):
            j = t * NB + b

            @pl.when(t > 0)
            def _():
                pltpu.make_async_copy(
                    ei_hbm.at[1, pl.ds(0, CHUNK)], didx[b], ssem[b]).wait()
            pltpu.async_copy(
                ei_hbm.at[1, pl.ds(ebase + j * CHUNK, CHUNK)], didx[b], isem[b])
        for b in range(NB):
            pltpu.make_async_copy(
                ei_hbm.at[1, pl.ds(0, CHUNK)], didx[b], isem[b]).wait()
            pltpu.async_copy(ones, acc.at[didx[b]], ssem[b], add=True)
        return 0
    lax.fori_loop(0, ngrp, _grp, 0)
    for b in range(NB):
        pltpu.make_async_copy(
            ei_hbm.at[1, pl.ds(0, CHUNK)], didx[b], ssem[b]).wait()
    plsc.subcore_barrier()
    pltpu.sync_copy(acc.at[pl.ds(s * RPT, RPT)],
                    degp_hbm.at[c, pl.ds(s * RPT, RPT)])


@functools.partial(
    pl.kernel,
    out_type=jax.ShapeDtypeStruct((NCORES, NPAD, F), jnp.float32),
    mesh=_mesh,
    scratch_types=[
        pltpu.VMEM((EPT,), jnp.int32),
        pltpu.VMEM((SCHUNK,), jnp.int32),
        pltpu.VMEM((SCHUNK,), jnp.int32),
        pltpu.VMEM((SCHUNK,), jnp.int32),
        pltpu.VMEM((SCHUNK,), jnp.int32),
        pltpu.VMEM((SCHUNK,), jnp.int32),
        pltpu.VMEM((SCHUNK,), jnp.int32),
        pltpu.VMEM((SCHUNK,), jnp.int32),
        pltpu.VMEM((SCHUNK,), jnp.int32),
        pltpu.VMEM((SCHUNK, F), jnp.float32),
        pltpu.VMEM((SCHUNK, F), jnp.float32),
        pltpu.VMEM((SCHUNK, F), jnp.float32),
        pltpu.VMEM((SCHUNK, F), jnp.float32),
        pltpu.VMEM((SCHUNK, F), jnp.float32),
        pltpu.VMEM((SCHUNK, F), jnp.float32),
        pltpu.VMEM((SCHUNK, F), jnp.float32),
        pltpu.VMEM((SCHUNK, F), jnp.float32),
        pltpu.VMEM_SHARED((NPAD, F), jnp.float32),
    ] + [pltpu.SemaphoreType.DMA] * 24,
)
def _scatter(g_hbm, ei_hbm, sout_hbm, sidx_all,
             d0, d1, d2, d3, d4, d5, d6, d7,
             r0, r1, r2, r3, r4, r5, r6, r7, acc,
             i0, i1, i2, i3, i4, i5, i6, i7,
             g0, g1, g2, g3, g4, g5, g6, g7,
             s0, s1, s2, s3, s4, s5, s6, s7):
    didx = (d0, d1, d2, d3, d4, d5, d6, d7)
    rows = (r0, r1, r2, r3, r4, r5, r6, r7)
    isem = (i0, i1, i2, i3, i4, i5, i6, i7)
    gsem = (g0, g1, g2, g3, g4, g5, g6, g7)
    ssem = (s0, s1, s2, s3, s4, s5, s6, s7)
    c = lax.axis_index("c")
    s = lax.axis_index("s")
    wid = c * NSUB + s

    def _zrow(i, _):
        for jj in range(F // 16):
            r0[i, pl.ds(jj * 16, 16)] = jnp.zeros((16,), jnp.float32)
        return 0
    lax.fori_loop(0, SCHUNK, _zrow, 0)
    for k in range(RPT // SCHUNK):
        pltpu.sync_copy(r0, acc.at[pl.ds(s * RPT + k * SCHUNK, SCHUNK)])
    plsc.subcore_barrier()

    ebase = wid * EPT
    last = wid == NTILES - 1
    pltpu.sync_copy(
        ei_hbm.at[0, pl.ds(jnp.where(last, E - EPT, ebase), EPT)], sidx_all)
    soff = jnp.where(last, EPT - LAST_REAL, 0)
    ngrp = jnp.where(last, LAST_REAL // SCHUNK // SNB, SNCHUNKS // SNB)

    def _grp(t, _):
        for b in range(SNB):
            j = t * SNB + b

            @pl.when(t > 0)
            def _():
                pltpu.make_async_copy(
                    g_hbm.at[pl.ds(0, SCHUNK)], rows[b], ssem[b]).wait()
            pltpu.async_copy(
                ei_hbm.at[1, pl.ds(ebase + j * SCHUNK, SCHUNK)], didx[b], isem[b])
            pltpu.async_copy(
                g_hbm.at[sidx_all.at[pl.ds(soff + j * SCHUNK, SCHUNK)]],
                rows[b], gsem[b])
        for b in range(SNB):
            j = t * SNB + b
            pltpu.make_async_copy(
                ei_hbm.at[1, pl.ds(0, SCHUNK)], didx[b], isem[b]).wait()
            pltpu.make_async_copy(
                g_hbm.at[pl.ds(0, SCHUNK)], rows[b], gsem[b]).wait()
            pltpu.async_copy(rows[b], acc.at[didx[b]], ssem[b], add=True)
        return 0
    lax.fori_loop(0, ngrp, _grp, 0)
    for b in range(SNB):
        pltpu.make_async_copy(
            g_hbm.at[pl.ds(0, SCHUNK)], rows[b], ssem[b]).wait()
    plsc.subcore_barrier()
    pltpu.sync_copy(acc.at[pl.ds(s * RPT, RPT)],
                    sout_hbm.at[c, pl.ds(s * RPT, RPT)])


def _dinv_block(degp_ref, i):
    d = degp_ref[0, pl.ds(i * BLK, BLK)] + degp_ref[1, pl.ds(i * BLK, BLK)]
    return lax.rsqrt(d + 1.0)


def _mm_scale_body(degp_ref, x_ref, w_ref, g_ref):
    i = pl.program_id(0)
    dinv = _dinv_block(degp_ref, i)
    h = jnp.dot(x_ref[...], w_ref[...], preferred_element_type=jnp.float32)
    g_ref[...] = h * dinv[:, None]


def _combine_mm_body(degp_ref, s_ref, g_ref, b_ref, w_ref, g2_ref):
    i = pl.program_id(0)
    dinv = _dinv_block(degp_ref, i)
    t = s_ref[0] + s_ref[1] + g_ref[...]
    z = jnp.maximum(t * dinv[:, None] + b_ref[...], 0.0)
    h2 = jnp.dot(z, w_ref[...], preferred_element_type=jnp.float32)
    g2_ref[...] = h2 * dinv[:, None]


def _final_body(degp_ref, s_ref, g_ref, b_ref, w_ref, fcb_ref, o_ref):
    i = pl.program_id(0)
    dinv = _dinv_block(degp_ref, i)
    t = s_ref[0] + s_ref[1] + g_ref[...]
    z = jnp.maximum(t * dinv[:, None] + b_ref[...], 0.0)
    o_ref[...] = (jnp.dot(z, w_ref[...], preferred_element_type=jnp.float32)
                  + fcb_ref[...])


_degp_spec = pl.BlockSpec((NCORES, NPAD), lambda i: (0, 0))
_row_spec = pl.BlockSpec((BLK, F), lambda i: (i, 0))
_s_spec = pl.BlockSpec((NCORES, BLK, F), lambda i: (0, i, 0))
_w_spec = pl.BlockSpec((F, F), lambda i: (0, 0))
_b_spec = pl.BlockSpec((1, F), lambda i: (0, 0))

_mm_scale = pl.pallas_call(
    _mm_scale_body,
    grid=(GRID,),
    in_specs=[_degp_spec, _row_spec, _w_spec],
    out_specs=_row_spec,
    out_shape=jax.ShapeDtypeStruct((NPAD, F), jnp.float32),
)

_combine_mm = pl.pallas_call(
    _combine_mm_body,
    grid=(GRID,),
    in_specs=[_degp_spec, _s_spec, _row_spec, _b_spec, _w_spec],
    out_specs=_row_spec,
    out_shape=jax.ShapeDtypeStruct((NPAD, F), jnp.float32),
)

_final = pl.pallas_call(
    _final_body,
    grid=(GRID,),
    in_specs=[_degp_spec, _s_spec, _row_spec, _b_spec,
              pl.BlockSpec((F, NCLS), lambda i: (0, 0)),
              pl.BlockSpec((1, NCLS), lambda i: (0, 0))],
    out_specs=pl.BlockSpec((BLK, NCLS), lambda i: (i, 0)),
    out_shape=jax.ShapeDtypeStruct((N, NCLS), jnp.float32),
)


def kernel(x, edge_index, W1, b1, W2, b2, fcW, fcb):
    degp = _hist(edge_index)
    g1 = _mm_scale(degp, x, W1)
    s1 = _scatter(g1, edge_index)
    g2 = _combine_mm(degp, s1, g1, b1.reshape(1, F), W2)
    s2 = _scatter(g2, edge_index)
    return _final(degp, s2, g2, b2.reshape(1, F), fcW, fcb.reshape(1, NCLS))
